# baseline (speedup 1.0000x reference)
import jax
import jax.numpy as jnp
from jax import lax
from jax.experimental import pallas as pl
from jax.experimental.pallas import tpu as pltpu

N_ROWS = 32


def _tc_dma_body(x_hbm, src_hbm, out_hbm, sem_x, sem_s):
    for i in range(1, N_ROWS):
        pltpu.make_async_copy(x_hbm.at[i], out_hbm.at[i], sem_x).start()
    pltpu.make_async_copy(src_hbm, out_hbm.at[0], sem_s).start()
    for i in range(1, N_ROWS):
        pltpu.make_async_copy(x_hbm.at[i], out_hbm.at[i], sem_x).wait()
    pltpu.make_async_copy(src_hbm, out_hbm.at[0], sem_s).wait()


def kernel(x, src):
    return pl.pallas_call(
        _tc_dma_body,
        out_shape=jax.ShapeDtypeStruct(x.shape, x.dtype),
        in_specs=[pl.BlockSpec(memory_space=pltpu.MemorySpace.HBM)] * 2,
        out_specs=pl.BlockSpec(memory_space=pltpu.MemorySpace.HBM),
        scratch_shapes=[pltpu.SemaphoreType.DMA, pltpu.SemaphoreType.DMA],
    )(x, src)


# speedup vs baseline: 30.8700x; 30.8700x over previous
import jax
import jax.numpy as jnp
from jax import lax
from jax.experimental import pallas as pl
from jax.experimental.pallas import tpu as pltpu

N_ROWS = 32
ROWS = 16384
COLS = 128
BM = 2048
C = ROWS // BM


def _tc_body(x_ref, src_ref, out_ref):
    i = pl.program_id(1)

    @pl.when(i == 0)
    def _():
        out_ref[0] = src_ref[...]

    @pl.when(i != 0)
    def _():
        out_ref[...] = x_ref[...]


def kernel(x, src):
    return pl.pallas_call(
        _tc_body,
        out_shape=jax.ShapeDtypeStruct(x.shape, x.dtype),
        grid=(C, N_ROWS),
        in_specs=[
            pl.BlockSpec(
                (1, BM, COLS),
                lambda j, i: (jnp.maximum(i, 1), j, 0),
            ),
            pl.BlockSpec((BM, COLS), lambda j, i: (j, 0)),
        ],
        out_specs=pl.BlockSpec((1, BM, COLS), lambda j, i: (i, j, 0)),
    )(x, src)


# speedup vs baseline: 31.1072x; 1.0077x over previous
import jax
import jax.numpy as jnp
from jax import lax
from jax.experimental import pallas as pl
from jax.experimental.pallas import tpu as pltpu

N_ROWS = 32
ROWS = 16384
COLS = 128
CH = 2048
PER_ROW = ROWS // CH
NCH = N_ROWS * PER_ROW
NBUF = 8


def _read(x_hbm, src_hbm, buf, sem, i):
    r = i // PER_ROW
    sl = pl.ds((i % PER_ROW) * CH, CH)

    @pl.when(r == 0)
    def _():
        pltpu.make_async_copy(src_hbm.at[sl], buf, sem).start()

    @pl.when(r != 0)
    def _():
        pltpu.make_async_copy(x_hbm.at[r, sl], buf, sem).start()


def _tc_body(x_hbm, src_hbm, out_hbm, *scratch):
    bufs = scratch[:NBUF]
    rsems = scratch[NBUF:2 * NBUF]
    wsems = scratch[2 * NBUF:]

    for b in range(NBUF):
        _read(x_hbm, src_hbm, bufs[b], rsems[b], b)

    def body(g, carry):
        for b in range(NBUF):
            i = g * NBUF + b
            r = i // PER_ROW
            sl = pl.ds((i % PER_ROW) * CH, CH)
            pltpu.make_async_copy(bufs[b], out_hbm.at[r, sl], rsems[b]).wait()
            pltpu.make_async_copy(bufs[b], out_hbm.at[r, sl], wsems[b]).start()
            nxt = i + NBUF

            @pl.when(nxt < NCH)
            def _():
                pltpu.make_async_copy(
                    bufs[b], out_hbm.at[r, sl], wsems[b]).wait()
                _read(x_hbm, src_hbm, bufs[b], rsems[b], nxt)
        return carry

    lax.fori_loop(0, NCH // NBUF, body, 0)
    for b in range(NBUF):
        i = NCH - NBUF + b
        r = i // PER_ROW
        sl = pl.ds((i % PER_ROW) * CH, CH)
        pltpu.make_async_copy(bufs[b], out_hbm.at[r, sl], wsems[b]).wait()


def kernel(x, src):
    return pl.pallas_call(
        _tc_body,
        out_shape=jax.ShapeDtypeStruct(x.shape, x.dtype),
        in_specs=[pl.BlockSpec(memory_space=pltpu.MemorySpace.HBM)] * 2,
        out_specs=pl.BlockSpec(memory_space=pltpu.MemorySpace.HBM),
        scratch_shapes=(
            [pltpu.VMEM((CH, COLS), jnp.float32) for _ in range(NBUF)]
            + [pltpu.SemaphoreType.DMA for _ in range(2 * NBUF)]
        ),
    )(x, src)


# speedup vs baseline: 39.8180x; 1.2800x over previous
import jax
import jax.numpy as jnp
from jax import lax
from jax.experimental import pallas as pl
from jax.experimental.pallas import tpu as pltpu
from jax.experimental.pallas import tpu_sc as plsc

ROWS = 16384
COLS = 128
CHUNK = 128
NBUF = 4
W = 2
K = NBUF - W
NCH = ROWS // CHUNK


def _sl(i):
    return pl.ds(i * CHUNK, CHUNK)


def _copy_pipeline(src_ref, dst_ref, bufs, rsems, wsems):
    for j in range(K):
        pltpu.make_async_copy(src_ref.at[_sl(j)], bufs[j], rsems[j]).start()

    def body(g, carry):
        for b in range(NBUF):
            i = g * NBUF + b
            pltpu.make_async_copy(src_ref.at[_sl(i)], bufs[b],
                                  rsems[b]).wait()
            pltpu.make_async_copy(bufs[b], dst_ref.at[_sl(i)],
                                  wsems[b]).start()
            bw = (b - W) % NBUF

            @pl.when(i >= W)
            def _():
                pltpu.make_async_copy(bufs[bw], dst_ref.at[_sl(i - W)],
                                      wsems[bw]).wait()

            br = (b + K) % NBUF

            @pl.when(i + K < NCH)
            def _():
                pltpu.make_async_copy(src_ref.at[_sl(i + K)], bufs[br],
                                      rsems[br]).start()
        return carry

    lax.fori_loop(0, NCH // NBUF, body, 0)
    for t in range(W):
        i = NCH - W + t
        b = i % NBUF
        pltpu.make_async_copy(bufs[b], dst_ref.at[_sl(i)], wsems[b]).wait()


def _sc_body(x_hbm, src_hbm, out_hbm, *scratch):
    c = lax.axis_index("c")
    s = lax.axis_index("s")
    w = s * 2 + c
    bufs = scratch[:NBUF]
    rsems = scratch[NBUF:2 * NBUF]
    wsems = scratch[2 * NBUF:]

    @pl.when(w == 0)
    def _():
        _copy_pipeline(src_hbm, out_hbm.at[0], bufs, rsems, wsems)

    @pl.when(w != 0)
    def _():
        _copy_pipeline(x_hbm.at[w], out_hbm.at[w], bufs, rsems, wsems)


def kernel(x, src):
    mesh = plsc.VectorSubcoreMesh(core_axis_name="c", subcore_axis_name="s")
    return pl.kernel(
        _sc_body,
        out_type=jax.ShapeDtypeStruct(x.shape, x.dtype),
        mesh=mesh,
        scratch_types=(
            [pltpu.VMEM((CHUNK, COLS), jnp.float32) for _ in range(NBUF)]
            + [pltpu.SemaphoreType.DMA for _ in range(2 * NBUF)]
        ),
    )(x, src)


# speedup vs baseline: 48.1474x; 1.2092x over previous
import jax
import jax.numpy as jnp
from jax import lax
from jax.experimental import pallas as pl
from jax.experimental.pallas import tpu as pltpu

N_ROWS = 32
ROWS = 16384
COLS = 128
CH = 2048
PER_ROW = ROWS // CH
NCH = N_ROWS * PER_ROW
NBUF = 8
W = 4
K = NBUF - W


def _read(x_hbm, src_hbm, buf, sem, i):
    r = i // PER_ROW
    sl = pl.ds((i % PER_ROW) * CH, CH)

    @pl.when(r == 0)
    def _():
        pltpu.make_async_copy(src_hbm.at[sl], buf, sem).start()

    @pl.when(r != 0)
    def _():
        pltpu.make_async_copy(x_hbm.at[r, sl], buf, sem).start()


def _wr(out_hbm, buf, sem, i):
    r = i // PER_ROW
    sl = pl.ds((i % PER_ROW) * CH, CH)
    return pltpu.make_async_copy(buf, out_hbm.at[r, sl], sem)


def _tc_body(x_hbm, src_hbm, out_hbm, *scratch):
    bufs = scratch[:NBUF]
    rsems = scratch[NBUF:2 * NBUF]
    wsems = scratch[2 * NBUF:]

    for j in range(K):
        _read(x_hbm, src_hbm, bufs[j], rsems[j], j)

    def body(g, carry):
        for b in range(NBUF):
            i = g * NBUF + b
            _wr(out_hbm, bufs[b], rsems[b], i).wait()
            _wr(out_hbm, bufs[b], wsems[b], i).start()
            bw = (b - W) % NBUF

            @pl.when(i >= W)
            def _():
                _wr(out_hbm, bufs[bw], wsems[bw], i - W).wait()

            br = (b + K) % NBUF

            @pl.when(i + K < NCH)
            def _():
                _read(x_hbm, src_hbm, bufs[br], rsems[br], i + K)
        return carry

    lax.fori_loop(0, NCH // NBUF, body, 0)
    for t in range(W):
        i = NCH - W + t
        _wr(out_hbm, bufs[i % NBUF], wsems[i % NBUF], i).wait()


def kernel(x, src):
    return pl.pallas_call(
        _tc_body,
        out_shape=jax.ShapeDtypeStruct(x.shape, x.dtype),
        in_specs=[pl.BlockSpec(memory_space=pltpu.MemorySpace.HBM)] * 2,
        out_specs=pl.BlockSpec(memory_space=pltpu.MemorySpace.HBM),
        scratch_shapes=(
            [pltpu.VMEM((CH, COLS), jnp.float32) for _ in range(NBUF)]
            + [pltpu.SemaphoreType.DMA for _ in range(2 * NBUF)]
        ),
    )(x, src)


# speedup vs baseline: 49.0451x; 1.0186x over previous
import jax
import jax.numpy as jnp
from jax import lax
from jax.experimental import pallas as pl
from jax.experimental.pallas import tpu as pltpu

N_ROWS = 32
ROWS = 16384
COLS = 128
CH = 2048
PER_ROW = ROWS // CH
NCH = N_ROWS * PER_ROW
NBUF = 16
W = 8
K = NBUF - W


def _read(x_hbm, src_hbm, buf, sem, i):
    r = i // PER_ROW
    sl = pl.ds((i % PER_ROW) * CH, CH)

    @pl.when(r == 0)
    def _():
        pltpu.make_async_copy(src_hbm.at[sl], buf, sem).start()

    @pl.when(r != 0)
    def _():
        pltpu.make_async_copy(x_hbm.at[r, sl], buf, sem).start()


def _wr(out_hbm, buf, sem, i):
    r = i // PER_ROW
    sl = pl.ds((i % PER_ROW) * CH, CH)
    return pltpu.make_async_copy(buf, out_hbm.at[r, sl], sem)


def _tc_body(x_hbm, src_hbm, out_hbm, *scratch):
    bufs = scratch[:NBUF]
    rsems = scratch[NBUF:2 * NBUF]
    wsems = scratch[2 * NBUF:]

    for j in range(K):
        _read(x_hbm, src_hbm, bufs[j], rsems[j], j)

    def body(g, carry):
        for b in range(NBUF):
            i = g * NBUF + b
            _wr(out_hbm, bufs[b], rsems[b], i).wait()
            _wr(out_hbm, bufs[b], wsems[b], i).start()
            bw = (b - W) % NBUF

            @pl.when(i >= W)
            def _():
                _wr(out_hbm, bufs[bw], wsems[bw], i - W).wait()

            br = (b + K) % NBUF

            @pl.when(i + K < NCH)
            def _():
                _read(x_hbm, src_hbm, bufs[br], rsems[br], i + K)
        return carry

    lax.fori_loop(0, NCH // NBUF, body, 0)
    for t in range(W):
        i = NCH - W + t
        _wr(out_hbm, bufs[i % NBUF], wsems[i % NBUF], i).wait()


def kernel(x, src):
    return pl.pallas_call(
        _tc_body,
        out_shape=jax.ShapeDtypeStruct(x.shape, x.dtype),
        in_specs=[pl.BlockSpec(memory_space=pltpu.MemorySpace.HBM)] * 2,
        out_specs=pl.BlockSpec(memory_space=pltpu.MemorySpace.HBM),
        scratch_shapes=(
            [pltpu.VMEM((CH, COLS), jnp.float32) for _ in range(NBUF)]
            + [pltpu.SemaphoreType.DMA for _ in range(2 * NBUF)]
        ),
    )(x, src)


# speedup vs baseline: 49.0794x; 1.0007x over previous
import jax
import jax.numpy as jnp
from jax import lax
from jax.experimental import pallas as pl
from jax.experimental.pallas import tpu as pltpu

N_ROWS = 32
ROWS = 16384
COLS = 128
CH = 4096
PER_ROW = ROWS // CH
NCH = N_ROWS * PER_ROW
NBUF = 16
W = 8
K = NBUF - W


def _read(x_hbm, src_hbm, buf, sem, i):
    r = i // PER_ROW
    sl = pl.ds((i % PER_ROW) * CH, CH)

    @pl.when(r == 0)
    def _():
        pltpu.make_async_copy(src_hbm.at[sl], buf, sem).start()

    @pl.when(r != 0)
    def _():
        pltpu.make_async_copy(x_hbm.at[r, sl], buf, sem).start()


def _wr(out_hbm, buf, sem, i):
    r = i // PER_ROW
    sl = pl.ds((i % PER_ROW) * CH, CH)
    return pltpu.make_async_copy(buf, out_hbm.at[r, sl], sem)


def _tc_body(x_hbm, src_hbm, out_hbm, *scratch):
    bufs = scratch[:NBUF]
    rsems = scratch[NBUF:2 * NBUF]
    wsems = scratch[2 * NBUF:]

    for j in range(K):
        _read(x_hbm, src_hbm, bufs[j], rsems[j], j)

    def body(g, carry):
        for b in range(NBUF):
            i = g * NBUF + b
            _wr(out_hbm, bufs[b], rsems[b], i).wait()
            _wr(out_hbm, bufs[b], wsems[b], i).start()
            bw = (b - W) % NBUF

            @pl.when(i >= W)
            def _():
                _wr(out_hbm, bufs[bw], wsems[bw], i - W).wait()

            br = (b + K) % NBUF

            @pl.when(i + K < NCH)
            def _():
                _read(x_hbm, src_hbm, bufs[br], rsems[br], i + K)
        return carry

    lax.fori_loop(0, NCH // NBUF, body, 0)
    for t in range(W):
        i = NCH - W + t
        _wr(out_hbm, bufs[i % NBUF], wsems[i % NBUF], i).wait()


def kernel(x, src):
    return pl.pallas_call(
        _tc_body,
        out_shape=jax.ShapeDtypeStruct(x.shape, x.dtype),
        in_specs=[pl.BlockSpec(memory_space=pltpu.MemorySpace.HBM)] * 2,
        out_specs=pl.BlockSpec(memory_space=pltpu.MemorySpace.HBM),
        scratch_shapes=(
            [pltpu.VMEM((CH, COLS), jnp.float32) for _ in range(NBUF)]
            + [pltpu.SemaphoreType.DMA for _ in range(2 * NBUF)]
        ),
    )(x, src)
